# baseline (speedup 1.0000x reference)
import functools

import jax
import jax.numpy as jnp
from jax import lax
from jax.experimental import pallas as pl
from jax.experimental.pallas import tpu as pltpu
from jax.experimental.pallas import tpu_sc as plsc

MASK_TAU = 0.5
L = 16
NC, NS = 2, 16
NW = NC * NS


def _sc_body(b_per_w, n_chunks, users_m, items_m, users, pos, neg, out,
             idx_u, idx_p, idx_n, rows_u, rows_p, rows_n, x_v, sem):
    wid = lax.axis_index("s") * NC + lax.axis_index("c")
    base = wid * b_per_w
    pltpu.sync_copy(users.at[pl.ds(base, b_per_w)], idx_u)
    pltpu.sync_copy(pos.at[pl.ds(base, b_per_w)], idx_p)
    pltpu.sync_copy(neg.at[pl.ds(base, b_per_w)], idx_n)
    copies = []
    for c in range(n_chunks):
        sl = pl.ds(c * 128, 128)
        copies.append(pltpu.async_copy(users_m.at[idx_u.at[sl]], rows_u.at[sl], sem))
        copies.append(pltpu.async_copy(items_m.at[idx_p.at[sl]], rows_p.at[sl], sem))
        copies.append(pltpu.async_copy(items_m.at[idx_n.at[sl]], rows_n.at[sl], sem))
    for cp in copies:
        cp.wait()

    lanes = lax.iota(jnp.int32, L)
    d_dim = rows_u.shape[1]

    @pl.loop(0, b_per_w // L)
    def _blk(b0):
        row = b0 * L + lanes
        acc = jnp.zeros((L,), jnp.float32)
        for d in range(d_dim):
            col = jnp.full((L,), d, jnp.int32)
            u = plsc.load_gather(rows_u, [row, col])
            p = plsc.load_gather(rows_p, [row, col])
            n = plsc.load_gather(rows_n, [row, col])
            acc = acc + u * (p - n)
        x_v[pl.ds(b0 * L, L)] = acc

    pltpu.sync_copy(x_v, out.at[pl.ds(base, b_per_w)])


def _sc_scores(users_m, items_m, users, pos, neg):
    b = users.shape[0]
    d = users_m.shape[1]
    b_per_w = b // NW
    n_chunks = b_per_w // 128
    mesh = plsc.VectorSubcoreMesh(core_axis_name="c", subcore_axis_name="s",
                                  num_cores=NC, num_subcores=NS)
    f = pl.kernel(
        functools.partial(_sc_body, b_per_w, n_chunks),
        out_type=jax.ShapeDtypeStruct((b,), jnp.float32),
        mesh=mesh,
        compiler_params=pltpu.CompilerParams(needs_layout_passes=False,
                                             use_tc_tiling_on_sc=False),
        scratch_types=[
            pltpu.VMEM((b_per_w,), jnp.int32),
            pltpu.VMEM((b_per_w,), jnp.int32),
            pltpu.VMEM((b_per_w,), jnp.int32),
            pltpu.VMEM((b_per_w, d), jnp.float32),
            pltpu.VMEM((b_per_w, d), jnp.float32),
            pltpu.VMEM((b_per_w, d), jnp.float32),
            pltpu.VMEM((b_per_w,), jnp.float32),
            pltpu.SemaphoreType.DMA,
        ],
    )
    return f(users_m, items_m, users, pos, neg)


def _tc_body(n_rows, b, au, aum, ai, aim, x, out, acc):
    i = pl.program_id(0)

    @pl.when(i == 0)
    def _():
        acc[0] = 0.0
        acc[1] = 0.0

    def cos_sum(a_ref, b_ref):
        a = a_ref[...]
        bb = b_ref[...]
        num = jnp.sum(a * bb, axis=1)
        den = (jnp.sqrt(jnp.sum(a * a, axis=1)) *
               jnp.sqrt(jnp.sum(bb * bb, axis=1)) + 1e-10)
        return jnp.sum(num / den)

    acc[0] += cos_sum(au, aum)
    acc[1] += cos_sum(ai, aim)

    @pl.when(i == pl.num_programs(0) - 1)
    def _():
        xx = x[...]
        bpr = jnp.sum(jnp.log(jax.nn.sigmoid(xx) + 1e-10))
        inv = 0.5 * (acc[0] / n_rows + acc[1] / n_rows)
        mf = -(bpr / b)
        out[0, 0] = -inv + MASK_TAU * mf


def _tc_loss(all_users, all_users_m, all_items, all_items_m, x2d):
    n = all_users.shape[0]
    d = all_users.shape[1]
    blk = 1000
    grid = n // blk
    b = x2d.shape[0] * x2d.shape[1]
    tbl_spec = pl.BlockSpec((blk, d), lambda i: (i, 0))
    return pl.pallas_call(
        functools.partial(_tc_body, float(n), float(b)),
        grid=(grid,),
        in_specs=[tbl_spec, tbl_spec, tbl_spec, tbl_spec,
                  pl.BlockSpec(x2d.shape, lambda i: (0, 0))],
        out_specs=pl.BlockSpec(memory_space=pltpu.SMEM),
        out_shape=jax.ShapeDtypeStruct((1, 1), jnp.float32),
        scratch_shapes=[pltpu.SMEM((2,), jnp.float32)],
        compiler_params=pltpu.CompilerParams(
            dimension_semantics=("arbitrary",)),
    )(all_users, all_users_m, all_items, all_items_m, x2d)


def kernel(all_users, all_items, all_users_m, all_items_m, users, pos_items, neg_items):
    x = _sc_scores(all_users_m, all_items_m, users, pos_items, neg_items)
    x2d = x.reshape(128, 128)
    loss = _tc_loss(all_users, all_users_m, all_items, all_items_m, x2d)
    return loss[0, 0]


# speedup vs baseline: 1.1913x; 1.1913x over previous
import functools

import jax
import jax.numpy as jnp
from jax import lax
from jax.experimental import pallas as pl
from jax.experimental.pallas import tpu as pltpu
from jax.experimental.pallas import tpu_sc as plsc

MASK_TAU = 0.5
L = 16
NC, NS = 2, 16
NW = NC * NS


def _sc_body(b_per_w, n_chunks, users_m, items_m, users, pos, neg, out,
             idx_u, idx_p, idx_n, rows_u, rows_p, rows_n, x_v, sem):
    wid = lax.axis_index("s") * NC + lax.axis_index("c")
    base = wid * b_per_w
    pltpu.sync_copy(users.at[pl.ds(base, b_per_w)], idx_u)
    pltpu.sync_copy(pos.at[pl.ds(base, b_per_w)], idx_p)
    pltpu.sync_copy(neg.at[pl.ds(base, b_per_w)], idx_n)
    copies = []
    for c in range(n_chunks):
        sl = pl.ds(c * 128, 128)
        copies.append(pltpu.async_copy(users_m.at[idx_u.at[sl]], rows_u.at[sl], sem))
        copies.append(pltpu.async_copy(items_m.at[idx_p.at[sl]], rows_p.at[sl], sem))
        copies.append(pltpu.async_copy(items_m.at[idx_n.at[sl]], rows_n.at[sl], sem))

    lanes = lax.iota(jnp.int32, L)
    d_dim = rows_u.shape[1]

    for c in range(n_chunks):
        for k in range(3):
            copies[3 * c + k].wait()

        @pl.loop(c * (128 // L), (c + 1) * (128 // L))
        def _blk(b0):
            row = b0 * L + lanes
            acc = [jnp.zeros((L,), jnp.float32) for _ in range(4)]
            for d in range(d_dim):
                col = jnp.full((L,), d, jnp.int32)
                u = plsc.load_gather(rows_u, [row, col])
                p = plsc.load_gather(rows_p, [row, col])
                n = plsc.load_gather(rows_n, [row, col])
                acc[d % 4] = acc[d % 4] + u * (p - n)
            x_v[pl.ds(b0 * L, L)] = (acc[0] + acc[1]) + (acc[2] + acc[3])

    pltpu.sync_copy(x_v, out.at[pl.ds(base, b_per_w)])


def _sc_scores(users_m, items_m, users, pos, neg):
    b = users.shape[0]
    d = users_m.shape[1]
    b_per_w = b // NW
    n_chunks = b_per_w // 128
    mesh = plsc.VectorSubcoreMesh(core_axis_name="c", subcore_axis_name="s",
                                  num_cores=NC, num_subcores=NS)
    f = pl.kernel(
        functools.partial(_sc_body, b_per_w, n_chunks),
        out_type=jax.ShapeDtypeStruct((b,), jnp.float32),
        mesh=mesh,
        compiler_params=pltpu.CompilerParams(needs_layout_passes=False,
                                             use_tc_tiling_on_sc=False),
        scratch_types=[
            pltpu.VMEM((b_per_w,), jnp.int32),
            pltpu.VMEM((b_per_w,), jnp.int32),
            pltpu.VMEM((b_per_w,), jnp.int32),
            pltpu.VMEM((b_per_w, d), jnp.float32),
            pltpu.VMEM((b_per_w, d), jnp.float32),
            pltpu.VMEM((b_per_w, d), jnp.float32),
            pltpu.VMEM((b_per_w,), jnp.float32),
            pltpu.SemaphoreType.DMA,
        ],
    )
    return f(users_m, items_m, users, pos, neg)


def _cos_body(au, aum, ai, aim, out, accv):
    i = pl.program_id(0)

    @pl.when(i == 0)
    def _():
        accv[...] = jnp.zeros_like(accv)

    d = au.shape[1]
    ones = jnp.ones((d, 128), jnp.float32)

    def pair(a_ref, b_ref):
        a = a_ref[...]
        b = b_ref[...]
        dims = (((1,), (0,)), ((), ()))
        num = lax.dot_general(a * b, ones, dims,
                              preferred_element_type=jnp.float32)
        na = lax.dot_general(a * a, ones, dims,
                             preferred_element_type=jnp.float32)
        nb = lax.dot_general(b * b, ones, dims,
                             preferred_element_type=jnp.float32)
        return num * lax.rsqrt(na * nb + 1e-20)

    accv[...] += pair(au, aum) + pair(ai, aim)

    @pl.when(i == pl.num_programs(0) - 1)
    def _():
        out[0, 0] = jnp.sum(accv[...]) * (1.0 / 128.0)


def _tc_cos(all_users, all_users_m, all_items, all_items_m):
    n = all_users.shape[0]
    d = all_users.shape[1]
    blk = 2000
    grid = n // blk
    tbl_spec = pl.BlockSpec((blk, d), lambda i: (i, 0))
    return pl.pallas_call(
        _cos_body,
        grid=(grid,),
        in_specs=[tbl_spec, tbl_spec, tbl_spec, tbl_spec],
        out_specs=pl.BlockSpec(memory_space=pltpu.SMEM),
        out_shape=jax.ShapeDtypeStruct((1, 1), jnp.float32),
        scratch_shapes=[pltpu.VMEM((blk, 128), jnp.float32)],
        compiler_params=pltpu.CompilerParams(
            dimension_semantics=("arbitrary",)),
    )(all_users, all_users_m, all_items, all_items_m)


def _final_body(n_rows, b, x, s, out):
    xx = x[...]
    bpr = jnp.sum(jnp.log(jax.nn.sigmoid(xx) + 1e-10))
    inv = 0.5 * (s[0, 0] / n_rows)
    mf = -(bpr / b)
    out[0, 0] = -inv + MASK_TAU * mf


def _tc_final(x2d, s, n_rows):
    b = x2d.shape[0] * x2d.shape[1]
    return pl.pallas_call(
        functools.partial(_final_body, float(n_rows), float(b)),
        in_specs=[pl.BlockSpec(x2d.shape, lambda: (0, 0)),
                  pl.BlockSpec(memory_space=pltpu.SMEM)],
        out_specs=pl.BlockSpec(memory_space=pltpu.SMEM),
        out_shape=jax.ShapeDtypeStruct((1, 1), jnp.float32),
    )(x2d, s)


def kernel(all_users, all_items, all_users_m, all_items_m, users, pos_items, neg_items):
    x = _sc_scores(all_users_m, all_items_m, users, pos_items, neg_items)
    s = _tc_cos(all_users, all_users_m, all_items, all_items_m)
    loss = _tc_final(x.reshape(128, 128), s, all_users.shape[0])
    return loss[0, 0]


# speedup vs baseline: 1.2733x; 1.0689x over previous
import functools

import jax
import jax.numpy as jnp
from jax import lax
from jax.experimental import pallas as pl
from jax.experimental.pallas import tpu as pltpu
from jax.experimental.pallas import tpu_sc as plsc

MASK_TAU = 0.5
L = 16
NC, NS = 2, 16
NW = NC * NS
BPW = 512
QUARTER = 128


def _sc_body(users_m, items_m, users, pos, neg, out,
             idx_u, idx_p, idx_n, idm_u, idm_p, idm_n,
             bu0, bp0, bn0, bu1, bp1, bn1, x_v, sem0, sem1):
    wid = lax.axis_index("s") * NC + lax.axis_index("c")
    base = wid * BPW
    pltpu.sync_copy(users.at[pl.ds(base, BPW)], idx_u)
    pltpu.sync_copy(pos.at[pl.ds(base, BPW)], idx_p)
    pltpu.sync_copy(neg.at[pl.ds(base, BPW)], idx_n)

    for j in range(BPW // L):
        sl = pl.ds(j * L, L)
        idm_u[sl] = lax.shift_right_logical(idx_u[sl], 1)
        idm_p[sl] = lax.shift_right_logical(idx_p[sl], 1)
        idm_n[sl] = lax.shift_right_logical(idx_n[sl], 1)

    bufs = [(bu0, bp0, bn0, sem0), (bu1, bp1, bn1, sem1)]

    def fire(q):
        bu, bp, bn, sem = bufs[q % 2]
        sl = pl.ds(q * QUARTER, QUARTER)
        return [
            pltpu.async_copy(users_m.at[idm_u.at[sl]], bu, sem),
            pltpu.async_copy(items_m.at[idm_p.at[sl]], bp, sem),
            pltpu.async_copy(items_m.at[idm_n.at[sl]], bn, sem),
        ]

    lanes = lax.iota(jnp.int32, L)
    cb = [jnp.bitwise_and(lanes + k, L - 1) for k in range(L)]

    n_q = BPW // QUARTER
    pend = fire(0)
    for q in range(n_q):
        nxt = fire(q + 1) if q + 1 < n_q else []
        for cp in pend:
            cp.wait()
        pend = nxt
        bu, bp, bn, _ = bufs[q % 2]

        @pl.loop(0, QUARTER // L)
        def _blk(b0):
            esl = pl.ds(q * QUARTER + b0 * L, L)
            par_u = jnp.bitwise_and(idx_u[esl], 1) * 64
            par_p = jnp.bitwise_and(idx_p[esl], 1) * 64
            par_n = jnp.bitwise_and(idx_n[esl], 1) * 64
            row = b0 * L + lanes
            acc = [jnp.zeros((L,), jnp.float32) for _ in range(4)]
            for g in range(4):
                for k in range(L):
                    col = cb[k] + (16 * g)
                    u = plsc.load_gather(bu, [row, par_u + col])
                    p = plsc.load_gather(bp, [row, par_p + col])
                    n = plsc.load_gather(bn, [row, par_n + col])
                    acc[k % 4] = acc[k % 4] + u * (p - n)
            x_v[esl] = (acc[0] + acc[1]) + (acc[2] + acc[3])

    pltpu.sync_copy(x_v, out.at[pl.ds(base, BPW)])


def _sc_scores(users_m2, items_m2, users, pos, neg):
    b = users.shape[0]
    mesh = plsc.VectorSubcoreMesh(core_axis_name="c", subcore_axis_name="s",
                                  num_cores=NC, num_subcores=NS)
    f = pl.kernel(
        _sc_body,
        out_type=jax.ShapeDtypeStruct((b,), jnp.float32),
        mesh=mesh,
        compiler_params=pltpu.CompilerParams(needs_layout_passes=False,
                                             use_tc_tiling_on_sc=True),
        scratch_types=[
            pltpu.VMEM((BPW,), jnp.int32),
            pltpu.VMEM((BPW,), jnp.int32),
            pltpu.VMEM((BPW,), jnp.int32),
            pltpu.VMEM((BPW,), jnp.int32),
            pltpu.VMEM((BPW,), jnp.int32),
            pltpu.VMEM((BPW,), jnp.int32),
            pltpu.VMEM((QUARTER, 128), jnp.float32),
            pltpu.VMEM((QUARTER, 128), jnp.float32),
            pltpu.VMEM((QUARTER, 128), jnp.float32),
            pltpu.VMEM((QUARTER, 128), jnp.float32),
            pltpu.VMEM((QUARTER, 128), jnp.float32),
            pltpu.VMEM((QUARTER, 128), jnp.float32),
            pltpu.VMEM((BPW,), jnp.float32),
            pltpu.SemaphoreType.DMA,
            pltpu.SemaphoreType.DMA,
        ],
    )
    return f(users_m2, items_m2, users, pos, neg)


def _cos_body(au, aum, ai, aim, out, accv):
    i = pl.program_id(0)

    @pl.when(i == 0)
    def _():
        accv[...] = jnp.zeros_like(accv)

    r = lax.broadcasted_iota(jnp.int32, (128, 128), 0)
    c = lax.broadcasted_iota(jnp.int32, (128, 128), 1)
    ones2 = jnp.where((r < 64) == (c < 64), 1.0, 0.0)

    def pair(a_ref, b_ref):
        a = a_ref[...]
        b = b_ref[...]
        dims = (((1,), (0,)), ((), ()))
        num = lax.dot_general(a * b, ones2, dims,
                              preferred_element_type=jnp.float32)
        na = lax.dot_general(a * a, ones2, dims,
                             preferred_element_type=jnp.float32)
        nb = lax.dot_general(b * b, ones2, dims,
                             preferred_element_type=jnp.float32)
        return num * lax.rsqrt(na * nb + 1e-20)

    accv[...] += pair(au, aum) + pair(ai, aim)

    @pl.when(i == pl.num_programs(0) - 1)
    def _():
        out[0, 0] = jnp.sum(accv[...]) * (1.0 / 64.0)


def _tc_cos(u2, um2, i2, im2):
    n2 = u2.shape[0]
    blk = 1000
    grid = n2 // blk
    tbl_spec = pl.BlockSpec((blk, 128), lambda i: (i, 0))
    return pl.pallas_call(
        _cos_body,
        grid=(grid,),
        in_specs=[tbl_spec, tbl_spec, tbl_spec, tbl_spec],
        out_specs=pl.BlockSpec(memory_space=pltpu.SMEM),
        out_shape=jax.ShapeDtypeStruct((1, 1), jnp.float32),
        scratch_shapes=[pltpu.VMEM((blk, 128), jnp.float32)],
        compiler_params=pltpu.CompilerParams(
            dimension_semantics=("arbitrary",)),
    )(u2, um2, i2, im2)


def _final_body(n_rows, b, x, s, out):
    xx = x[...]
    bpr = jnp.sum(jnp.log(jax.nn.sigmoid(xx) + 1e-10))
    inv = 0.5 * (s[0, 0] / n_rows)
    mf = -(bpr / b)
    out[0, 0] = -inv + MASK_TAU * mf


def _tc_final(x2d, s, n_rows):
    b = x2d.shape[0] * x2d.shape[1]
    return pl.pallas_call(
        functools.partial(_final_body, float(n_rows), float(b)),
        in_specs=[pl.BlockSpec(x2d.shape, lambda: (0, 0)),
                  pl.BlockSpec(memory_space=pltpu.SMEM)],
        out_specs=pl.BlockSpec(memory_space=pltpu.SMEM),
        out_shape=jax.ShapeDtypeStruct((1, 1), jnp.float32),
    )(x2d, s)


def kernel(all_users, all_items, all_users_m, all_items_m, users, pos_items, neg_items):
    n = all_users.shape[0]
    u2 = all_users.reshape(n // 2, 128)
    i2 = all_items.reshape(n // 2, 128)
    um2 = all_users_m.reshape(n // 2, 128)
    im2 = all_items_m.reshape(n // 2, 128)
    x = _sc_scores(um2, im2, users, pos_items, neg_items)
    s = _tc_cos(u2, um2, i2, im2)
    loss = _tc_final(x.reshape(128, 128), s, n)
    return loss[0, 0]


# speedup vs baseline: 2.2323x; 1.7532x over previous
import functools

import jax
import jax.numpy as jnp
from jax import lax
from jax.experimental import pallas as pl
from jax.experimental.pallas import tpu as pltpu
from jax.experimental.pallas import tpu_sc as plsc

MASK_TAU = 0.5
L = 16
NC, NS = 2, 16
NW = NC * NS
BPW = 512
QUARTER = 128


def _sc_body(users_m, items_m, users, pos, neg, out,
             idx_u, idx_p, idx_n, bu0, bp0, bn0, bu1, bp1, bn1,
             x_v, sem0, sem1):
    wid = lax.axis_index("s") * NC + lax.axis_index("c")
    base = wid * BPW
    pltpu.sync_copy(users.at[pl.ds(base, BPW)], idx_u)
    pltpu.sync_copy(pos.at[pl.ds(base, BPW)], idx_p)
    pltpu.sync_copy(neg.at[pl.ds(base, BPW)], idx_n)

    bufs = [(bu0, bp0, bn0, sem0), (bu1, bp1, bn1, sem1)]

    def fire(q):
        bu, bp, bn, sem = bufs[q % 2]
        sl = pl.ds(q * QUARTER, QUARTER)
        return [
            pltpu.async_copy(users_m.at[idx_u.at[sl]], bu, sem),
            pltpu.async_copy(items_m.at[idx_p.at[sl]], bp, sem),
            pltpu.async_copy(items_m.at[idx_n.at[sl]], bn, sem),
        ]

    lanes = lax.iota(jnp.int32, L)
    cb = [jnp.bitwise_and(lanes + k, L - 1) for k in range(L)]

    n_q = BPW // QUARTER
    pend = fire(0)
    for q in range(n_q):
        nxt = fire(q + 1) if q + 1 < n_q else []
        for cp in pend:
            cp.wait()
        pend = nxt
        bu, bp, bn, _ = bufs[q % 2]

        @pl.loop(0, QUARTER // L)
        def _blk(b0):
            row = b0 * L + lanes
            acc = [jnp.zeros((L,), jnp.float32) for _ in range(4)]
            for g in range(4):
                for k in range(L):
                    col = cb[k] + (16 * g)
                    u = plsc.load_gather(bu, [row, col])
                    p = plsc.load_gather(bp, [row, col])
                    n = plsc.load_gather(bn, [row, col])
                    acc[k % 4] = acc[k % 4] + u * (p - n)
            x_v[pl.ds(q * QUARTER + b0 * L, L)] = (
                (acc[0] + acc[1]) + (acc[2] + acc[3]))

    pltpu.sync_copy(x_v, out.at[pl.ds(base, BPW)])


def _sc_scores(users_m, items_m, users, pos, neg):
    b = users.shape[0]
    d = users_m.shape[1]
    mesh = plsc.VectorSubcoreMesh(core_axis_name="c", subcore_axis_name="s",
                                  num_cores=NC, num_subcores=NS)
    f = pl.kernel(
        _sc_body,
        out_type=jax.ShapeDtypeStruct((b,), jnp.float32),
        mesh=mesh,
        compiler_params=pltpu.CompilerParams(needs_layout_passes=False,
                                             use_tc_tiling_on_sc=False),
        scratch_types=[
            pltpu.VMEM((BPW,), jnp.int32),
            pltpu.VMEM((BPW,), jnp.int32),
            pltpu.VMEM((BPW,), jnp.int32),
            pltpu.VMEM((QUARTER, d), jnp.float32),
            pltpu.VMEM((QUARTER, d), jnp.float32),
            pltpu.VMEM((QUARTER, d), jnp.float32),
            pltpu.VMEM((QUARTER, d), jnp.float32),
            pltpu.VMEM((QUARTER, d), jnp.float32),
            pltpu.VMEM((QUARTER, d), jnp.float32),
            pltpu.VMEM((BPW,), jnp.float32),
            pltpu.SemaphoreType.DMA,
            pltpu.SemaphoreType.DMA,
        ],
    )
    return f(users_m, items_m, users, pos, neg)


def _cos_body(n, chunk, au, aum, ai, aim, out, acc):
    i = pl.program_id(0)

    @pl.when(i == 0)
    def _():
        acc[0] = 0.0

    valid = (i * chunk + lax.broadcasted_iota(jnp.int32, (chunk,), 0)) < n

    def pair(a_ref, b_ref):
        a = a_ref[...]
        b = b_ref[...]
        num = jnp.sum(a * b, axis=0)
        na = jnp.sum(a * a, axis=0)
        nb = jnp.sum(b * b, axis=0)
        ratio = num * lax.rsqrt(na * nb + 1e-20)
        return jnp.sum(jnp.where(valid, ratio, 0.0))

    acc[0] += pair(au, aum) + pair(ai, aim)

    @pl.when(i == pl.num_programs(0) - 1)
    def _():
        out[0, 0] = acc[0]


def _tc_cos(ut, umt, it_, imt):
    d, n = ut.shape
    chunk = 8192
    grid = (n + chunk - 1) // chunk
    tbl_spec = pl.BlockSpec((d, chunk), lambda i: (0, i))
    return pl.pallas_call(
        functools.partial(_cos_body, n, chunk),
        grid=(grid,),
        in_specs=[tbl_spec, tbl_spec, tbl_spec, tbl_spec],
        out_specs=pl.BlockSpec(memory_space=pltpu.SMEM),
        out_shape=jax.ShapeDtypeStruct((1, 1), jnp.float32),
        scratch_shapes=[pltpu.SMEM((1,), jnp.float32)],
        compiler_params=pltpu.CompilerParams(
            dimension_semantics=("arbitrary",)),
    )(ut, umt, it_, imt)


def _final_body(n_rows, b, x, s, out):
    xx = x[...]
    bpr = jnp.sum(jnp.log(jax.nn.sigmoid(xx) + 1e-10))
    inv = 0.5 * (s[0, 0] / n_rows)
    mf = -(bpr / b)
    out[0, 0] = -inv + MASK_TAU * mf


def _tc_final(x2d, s, n_rows):
    b = x2d.shape[0] * x2d.shape[1]
    return pl.pallas_call(
        functools.partial(_final_body, float(n_rows), float(b)),
        in_specs=[pl.BlockSpec(x2d.shape, lambda: (0, 0)),
                  pl.BlockSpec(memory_space=pltpu.SMEM)],
        out_specs=pl.BlockSpec(memory_space=pltpu.SMEM),
        out_shape=jax.ShapeDtypeStruct((1, 1), jnp.float32),
    )(x2d, s)


def kernel(all_users, all_items, all_users_m, all_items_m, users, pos_items, neg_items):
    n = all_users.shape[0]
    x = _sc_scores(all_users_m, all_items_m, users, pos_items, neg_items)
    s = _tc_cos(jnp.swapaxes(all_users, 0, 1), jnp.swapaxes(all_users_m, 0, 1),
                jnp.swapaxes(all_items, 0, 1), jnp.swapaxes(all_items_m, 0, 1))
    loss = _tc_final(x.reshape(128, 128), s, n)
    return loss[0, 0]
